# baseline (device time: 197365 ns/iter reference)
import jax
import jax.numpy as jnp
import numpy as np
from jax import lax
from jax.experimental import pallas as pl
from jax.experimental.pallas import tpu as pltpu

N_DEV = 32

_RING = np.array(
    [0, 1, 2, 5, 6, 7, 4, 3,
     11, 12, 15, 14, 13, 10, 9,
     17, 18, 21, 22, 23, 20, 19,
     27, 28, 31, 30, 29, 26, 25, 24,
     16, 8],
    dtype=np.int32,
)
_INV = np.argsort(_RING).astype(np.int32)

N_PIECES = 4
FWD_PIECES = (0, 1)
BWD_PIECES = (2, 3)


def kernel(x, w_mat):
    m_per, k = x.shape
    n = w_mat.shape[1]
    m_tot = N_DEV * m_per

    def body(ring_ref, inv_ref, x_ref, w_ref, out_ref, gx_ref, y_ref,
             out_stage, amax_ref, ring_send_sems, ring_recv_sems,
             sc_send_sems, sc_recv_sem, cp_sems):
        my = lax.axis_index("i")
        r = inv_ref[my]
        left = ring_ref[jnp.mod(r - 1, N_DEV)]
        right = ring_ref[jnp.mod(r + 1, N_DEV)]

        barrier_sem = pltpu.get_barrier_semaphore()
        pl.semaphore_signal(barrier_sem, inc=1, device_id=(left,),
                            device_id_type=pl.DeviceIdType.MESH)
        pl.semaphore_signal(barrier_sem, inc=1, device_id=(right,),
                            device_id_type=pl.DeviceIdType.MESH)
        pl.semaphore_wait(barrier_sem, 2)

        hm = m_per // N_PIECES

        def mk_send(origin, half, tgt, dir_idx):
            return pltpu.make_async_remote_copy(
                src_ref=gx_ref.at[origin, pl.ds(half * hm, hm)],
                dst_ref=gx_ref.at[origin, pl.ds(half * hm, hm)],
                send_sem=ring_send_sems.at[dir_idx, half],
                recv_sem=ring_recv_sems.at[origin, half],
                device_id=(tgt,),
                device_id_type=pl.DeviceIdType.MESH,
            )

        def wait_half(origin, half):
            pltpu.make_async_remote_copy(
                src_ref=gx_ref.at[origin, pl.ds(half * hm, hm)],
                dst_ref=gx_ref.at[origin, pl.ds(half * hm, hm)],
                send_sem=ring_send_sems.at[0, half],
                recv_sem=ring_recv_sems.at[origin, half],
                device_id=(my,),
                device_id_type=pl.DeviceIdType.MESH,
            ).wait_recv()

        fwd = [None] * N_PIECES
        bwd = [None] * N_PIECES
        for p in range(N_PIECES):
            gx_ref[pl.ds(my, 1), pl.ds(p * hm, hm)] = (
                x_ref[pl.ds(p * hm, hm), :].astype(jnp.bfloat16)
                .reshape(1, hm, k))
            fwd[p] = mk_send(my, p, right, 0)
            fwd[p].start()
            bwd[p] = mk_send(my, p, left, 1)
            bwd[p].start()

        w = w_ref[...].astype(jnp.bfloat16)
        xb0 = gx_ref[pl.ds(my, 1)].reshape(m_per, k)
        y0 = jnp.maximum(
            jnp.dot(xb0, w, preferred_element_type=jnp.float32), 0.0)
        y_ref[pl.ds(my * m_per, m_per), :] = y0
        amax = jnp.max(y0)

        def gemm_rows(origin, lo, nrows, amax):
            xb = gx_ref[pl.ds(origin, 1), pl.ds(lo, nrows)].reshape(nrows, k)
            yb = jnp.maximum(
                jnp.dot(xb, w, preferred_element_type=jnp.float32), 0.0)
            y_ref[pl.ds(origin * m_per + lo, nrows), :] = yb
            return jnp.maximum(amax, jnp.max(yb))

        def gemm_chunk(origin, amax):
            return gemm_rows(origin, 0, m_per, amax)

        for h in range(16):
            rf = ring_ref[jnp.mod(r - h - 1, N_DEV)]
            rb = ring_ref[jnp.mod(r + h + 1, N_DEV)]
            f_recv = range(N_PIECES) if h <= 14 else FWD_PIECES
            b_recv = range(N_PIECES) if h <= 14 else BWD_PIECES
            f_fw = range(N_PIECES) if h <= 13 else (FWD_PIECES if h == 14 else ())
            b_fw = range(N_PIECES) if h <= 13 else (BWD_PIECES if h == 14 else ())
            for p in range(N_PIECES):
                if p in f_recv:
                    wait_half(rf, p)
                    if p in f_fw:
                        fwd[p].wait_send()
                        fwd[p] = mk_send(rf, p, right, 0)
                        fwd[p].start()
                if h == 15 and p == FWD_PIECES[-1]:
                    amax = gemm_rows(rf, 0, len(FWD_PIECES) * hm, amax)
                if p in b_recv:
                    wait_half(rb, p)
                    if p in b_fw:
                        bwd[p].wait_send()
                        bwd[p] = mk_send(rb, p, left, 1)
                        bwd[p].start()
            if h <= 14:
                amax = gemm_chunk(rf, amax)
                amax = gemm_chunk(rb, amax)
            else:
                amax = gemm_rows(rf, len(FWD_PIECES) * hm,
                                 len(BWD_PIECES) * hm, amax)
        for d in (*fwd, *bwd):
            d.wait_send()

        amax_ref[pl.ds(my, 1)] = jnp.full((1, 128), amax, jnp.float32)
        descs = []
        for d in range(1, N_DEV):
            tgt = jnp.mod(my + d, N_DEV)
            s = pltpu.make_async_remote_copy(
                src_ref=amax_ref.at[my],
                dst_ref=amax_ref.at[my],
                send_sem=sc_send_sems.at[d - 1],
                recv_sem=sc_recv_sem,
                device_id=(tgt,),
                device_id_type=pl.DeviceIdType.MESH,
            )
            s.start()
            descs.append(s)
        for d in range(1, N_DEV):
            src = jnp.mod(my + d, N_DEV)
            pltpu.make_async_remote_copy(
                src_ref=amax_ref.at[src],
                dst_ref=amax_ref.at[src],
                send_sem=sc_send_sems.at[0],
                recv_sem=sc_recv_sem,
                device_id=(my,),
                device_id_type=pl.DeviceIdType.MESH,
            ).wait_recv()

        amax_g = jnp.maximum(jnp.max(amax_ref[...]), 1e-30)
        scale = amax_g / 448.0
        inv_scale = 448.0 / amax_g

        n_eb = 4
        eb = m_tot // n_eb
        cps = []
        for b in range(n_eb):
            yb = y_ref[pl.ds(b * eb, eb), :]
            qb = (yb * inv_scale).astype(jnp.float8_e4m3fn).astype(jnp.float32)
            out_stage[pl.ds(b * eb, eb), :] = (qb * scale).astype(jnp.bfloat16)
            cp = pltpu.make_async_copy(
                out_stage.at[pl.ds(b * eb, eb)],
                out_ref.at[pl.ds(b * eb, eb)],
                cp_sems.at[b],
            )
            cp.start()
            cps.append(cp)
        for cp in cps:
            cp.wait()
        for s in descs:
            s.wait_send()

    ring = jnp.asarray(_RING)
    inv = jnp.asarray(_INV)

    return pl.pallas_call(
        body,
        out_shape=jax.ShapeDtypeStruct((m_tot, n), jnp.bfloat16),
        in_specs=[
            pl.BlockSpec(memory_space=pltpu.SMEM),
            pl.BlockSpec(memory_space=pltpu.SMEM),
            pl.BlockSpec(memory_space=pltpu.VMEM),
            pl.BlockSpec(memory_space=pltpu.VMEM),
        ],
        out_specs=pl.BlockSpec(memory_space=pl.ANY),
        scratch_shapes=[
            pltpu.VMEM((N_DEV, m_per, k), jnp.bfloat16),
            pltpu.VMEM((m_tot, n), jnp.float32),
            pltpu.VMEM((m_tot, n), jnp.bfloat16),
            pltpu.VMEM((N_DEV, 128), jnp.float32),
            pltpu.SemaphoreType.DMA((2, N_PIECES)),
            pltpu.SemaphoreType.DMA((N_DEV, N_PIECES)),
            pltpu.SemaphoreType.DMA((N_DEV - 1,)),
            pltpu.SemaphoreType.DMA,
            pltpu.SemaphoreType.DMA((4,)),
        ],
        compiler_params=pltpu.CompilerParams(
            collective_id=0,
            vmem_limit_bytes=100 * 1024 * 1024,
        ),
    )(ring, inv, x, w_mat)
